# baseline (speedup 1.0000x reference)
import dataclasses
import functools

import jax
import jax.numpy as jnp
from jax import lax
from jax.experimental import pallas as pl
from jax.experimental.pallas import tpu as pltpu
from jax.experimental.pallas import tpu_sc as plsc

N = 10000
E = 320000
F = 128
K = 25
KS = 5

NPAD = 10240
NB = 40
CH_E = 32
NTILES = 32
CHUNKS = 316
EPT = CH_E * CHUNKS
EPAD = EPT * NTILES
ROWS = EPAD // 32
NAGG = 10240
NPS = NAGG // 16
DROWS = NAGG // 128


def _mm_body(f_ref, w_ref, o_ref):
    o_ref[...] = jnp.dot(f_ref[...], w_ref[...],
                         preferred_element_type=jnp.float32)


def _norm_body(p_ref, d_ref, b_ref, o_ref):
    msg = p_ref[0] + p_ref[1]
    deg = jnp.sum(d_ref[...], axis=0)
    o_ref[...] = msg / jnp.maximum(deg, 1.0) + b_ref[...]


def _sc_edge_kernel(table, meta, zeros, out, degs,
                    meta_v0, meta_v1, dst_v0, dst_v1, idx_v0, idx_v1,
                    rows_v0, rows_v1, msg_v, wbuf0, wbuf1, deg_v, agg_sh,
                    sg0, sg1, sm0, sm1):
    meta_v = (meta_v0, meta_v1)
    dst_v = (dst_v0, dst_v1)
    idx_v = (idx_v0, idx_v1)
    rows_v = (rows_v0, rows_v1)
    wbuf = (wbuf0, wbuf1)
    sem_g = (sg0, sg1)
    sem_m = (sm0, sm1)

    cid = lax.axis_index("c")
    sid = lax.axis_index("s")
    w = sid * 2 + cid
    mrow = w * CHUNKS

    lane = lax.iota(jnp.int32, 16)

    pltpu.sync_copy(zeros, agg_sh.at[pl.ds(sid * NPS, NPS)])
    pltpu.sync_copy(zeros.at[pl.ds(0, DROWS)], deg_v)
    plsc.subcore_barrier()

    def basis(b, mv, iv, dv, wv):
        for h in range(2):
            src = mv[pl.ds(16 * h, 16)]
            dv[pl.ds(16 * h, 16)] = mv[pl.ds(32 + 16 * h, 16)]
            wd = []
            idd = []
            for d in range(2):
                p = plsc.bitcast(mv[pl.ds(64 + 32 * d + 16 * h, 16)],
                                 jnp.float32)
                v = jnp.clip(p * (KS - 1), 0.0, KS - 1 - 1e-6)
                i0 = v.astype(jnp.int32)
                fr = v - i0.astype(jnp.float32)
                i1 = jnp.minimum(i0 + 1, KS - 1)
                wd.append((1.0 - fr, fr))
                idd.append((i0, i1))
            eid = (w * EPT + b * CH_E + 16 * h) + lane
            m = jnp.where(eid < E, 1.0, 0.0).astype(jnp.float32)
            wv[pl.ds(16 * (5 * h + 4), 16)] = m
            for s in range(4):
                ws = wd[0][s & 1] * wd[1][(s >> 1) & 1] * m
                wv[pl.ds(16 * (5 * h + s), 16)] = ws
                ki = idd[0][s & 1] * KS + idd[1][(s >> 1) & 1]
                plsc.store_scatter(iv, [lane * 4 + (64 * h + s)],
                                   src * K + ki)

    def compute(B):
        rv, wv, dv = rows_v[B], wbuf[B], dst_v[B]
        wvecs = [[wv[pl.ds(16 * (5 * h + s), 16)] for s in range(4)]
                 for h in range(2)]
        rbase = [lane * 4 + 64 * h for h in range(2)]
        evecs = [lane + 16 * h for h in range(2)]

        @pl.loop(0, F)
        def _(f):
            fvec = lane * 0 + f
            for h in range(2):
                acc = None
                for s in range(4):
                    g = plsc.load_gather(rv, [rbase[h] + s, fvec])
                    t = g * wvecs[h][s]
                    acc = t if acc is None else acc + t
                plsc.store_scatter(msg_v, [evecs[h], fvec], acc)

        @pl.loop(0, 2)
        def _(hh):
            dvec = dv[pl.ds(16 * hh, 16)]
            mvec = wv[pl.ds(80 * hh + 64, 16)]
            for le in range(16):
                d = dvec[le]
                dr = lax.shift_right_logical(d, 7)
                dbase = lax.bitwise_and(d, 0x70)
                dlane = lax.bitwise_and(d, 0xF)
                sl_d = pl.ds(dbase, 16)
                deg_v[dr, sl_d] = deg_v[dr, sl_d] + jnp.where(
                    lane == dlane, mvec[le], 0.0)

    def body(b, B):
        B2 = 1 - B

        @pl.when(b + 1 < CHUNKS)
        def _():
            pltpu.make_async_copy(meta.at[mrow + b + 1], meta_v[B2],
                                  sem_m[B2]).wait()
            basis(b + 1, meta_v[B2], idx_v[B2], dst_v[B2], wbuf[B2])
            pltpu.async_copy(table.at[idx_v[B2]], rows_v[B2], sem_g[B2])

        @pl.when(b + 2 < CHUNKS)
        def _():
            pltpu.async_copy(meta.at[mrow + b + 2], meta_v[B], sem_m[B])

        pltpu.make_async_copy(table.at[idx_v[B]], rows_v[B],
                              sem_g[B]).wait()
        compute(B)
        pltpu.sync_copy(msg_v, agg_sh.at[dst_v[B]], add=True)

    pltpu.sync_copy(meta.at[mrow], meta_v[0])
    basis(0, meta_v[0], idx_v[0], dst_v[0], wbuf[0])
    pltpu.async_copy(table.at[idx_v[0]], rows_v[0], sem_g[0])
    pltpu.async_copy(meta.at[mrow + 1], meta_v[1], sem_m[1])

    @pl.loop(0, CHUNKS // 2)
    def _(g):
        body(2 * g, 0)
        body(2 * g + 1, 1)

    degs_out = degs.at[w]
    pltpu.sync_copy(deg_v, degs_out)
    plsc.subcore_barrier()
    pltpu.sync_copy(agg_sh.at[pl.ds(sid * NPS, NPS)],
                    out.at[cid, pl.ds(sid * NPS, NPS)])


def kernel(features, edge_index, pseudo, weight, bias):
    f32 = jnp.float32

    feat_pad = jnp.pad(features, ((0, NPAD - N), (0, 0)))
    w2 = jnp.transpose(weight, (1, 0, 2)).reshape(F, K * F)

    pad = EPAD - E
    src2 = jnp.pad(edge_index[0], (0, pad)).reshape(ROWS, 32)
    dst2 = jnp.pad(edge_index[1], (0, pad)).reshape(ROWS, 32)
    p0 = lax.bitcast_convert_type(
        jnp.pad(pseudo[:, 0], (0, pad)).reshape(ROWS, 32), jnp.int32)
    p1 = lax.bitcast_convert_type(
        jnp.pad(pseudo[:, 1], (0, pad)).reshape(ROWS, 32), jnp.int32)
    meta = jnp.concatenate([src2, dst2, p0, p1], axis=1)
    zeros = jnp.zeros((NPS, F), f32)

    mm = pl.pallas_call(
        _mm_body,
        grid=(NB,),
        in_specs=[pl.BlockSpec((NPAD // NB, F), lambda m: (m, 0)),
                  pl.BlockSpec((F, K * F), lambda m: (0, 0))],
        out_specs=pl.BlockSpec((NPAD // NB, K * F), lambda m: (m, 0)),
        out_shape=jax.ShapeDtypeStruct((NPAD, K * F), f32),
    )
    table = mm(feat_pad, w2).reshape(NPAD * K, F)

    mesh = plsc.VectorSubcoreMesh(core_axis_name="c", subcore_axis_name="s")
    cp = pltpu.CompilerParams()
    if "needs_layout_passes" in pltpu.CompilerParams.__dataclass_fields__:
        cp = dataclasses.replace(cp, needs_layout_passes=False)
    sc = pl.kernel(
        _sc_edge_kernel,
        mesh=mesh,
        out_type=[jax.ShapeDtypeStruct((2, NAGG, F), f32),
                  jax.ShapeDtypeStruct((NTILES, DROWS, 128), f32)],
        scratch_types=(
            [pltpu.VMEM((128,), jnp.int32)] * 2
            + [pltpu.VMEM((CH_E,), jnp.int32)] * 2
            + [pltpu.VMEM((128,), jnp.int32)] * 2
            + [pltpu.VMEM((128, F), f32)] * 2
            + [pltpu.VMEM((CH_E, F), f32)]
            + [pltpu.VMEM((160,), f32)] * 2
            + [pltpu.VMEM((DROWS, 128), f32),
               pltpu.VMEM_SHARED((NAGG, F), f32)]
            + [pltpu.SemaphoreType.DMA] * 4
        ),
        compiler_params=cp,
    )
    parts, degp = sc(table, meta, zeros)
    degf = degp.reshape(NTILES, NAGG, 1)

    norm = pl.pallas_call(
        _norm_body,
        grid=(10,),
        in_specs=[pl.BlockSpec((2, N // 10, F), lambda i: (0, i, 0)),
                  pl.BlockSpec((NTILES, N // 10, 1), lambda i: (0, i, 0)),
                  pl.BlockSpec((1, F), lambda i: (0, 0))],
        out_specs=pl.BlockSpec((N // 10, F), lambda i: (i, 0)),
        out_shape=jax.ShapeDtypeStruct((N, F), f32),
    )
    return norm(parts, degf, bias.reshape(1, F))


# speedup vs baseline: 2.9970x; 2.9970x over previous
import dataclasses
import functools

import jax
import jax.numpy as jnp
from jax import lax
from jax.experimental import pallas as pl
from jax.experimental.pallas import tpu as pltpu
from jax.experimental.pallas import tpu_sc as plsc

N = 10000
E = 320000
F = 128
K = 25
KS = 5

NPAD = 10240
NB = 40
CH_E = 32
NTILES = 32
CHUNKS = 316
EPT = CH_E * CHUNKS
EPAD = EPT * NTILES
ROWS = EPAD // 32
NAGG = 10240
NPS = NAGG // 16
DROWS = NAGG // 128


def _mm_body(f_ref, w_ref, o_ref):
    o_ref[...] = jnp.dot(f_ref[...], w_ref[...],
                         preferred_element_type=jnp.float32)


def _norm_body(p_ref, d_ref, b_ref, o_ref):
    msg = p_ref[0] + p_ref[1]
    deg = jnp.sum(d_ref[...], axis=0)
    o_ref[...] = msg / jnp.maximum(deg, 1.0) + b_ref[...]


def _sc_edge_kernel(table, meta, zeros, out, degs,
                    meta_v0, meta_v1, dst_v0, dst_v1, idx_v0, idx_v1,
                    rows_v0, rows_v1, msg_v, deg_v, agg_sh,
                    sg0, sg1, sm0, sm1):
    meta_v = (meta_v0, meta_v1)
    dst_v = (dst_v0, dst_v1)
    idx_v = (idx_v0, idx_v1)
    rows_v = (rows_v0, rows_v1)
    sem_g = (sg0, sg1)
    sem_m = (sm0, sm1)

    cid = lax.axis_index("c")
    sid = lax.axis_index("s")
    w = sid * 2 + cid
    mrow = w * CHUNKS

    lane = lax.iota(jnp.int32, 16)

    pltpu.sync_copy(zeros, agg_sh.at[pl.ds(sid * NPS, NPS)])
    pltpu.sync_copy(zeros.at[pl.ds(0, DROWS)], deg_v)
    plsc.subcore_barrier()

    def spline(b, mv, h):
        wd = []
        idd = []
        for d in range(2):
            p = plsc.bitcast(mv[pl.ds(64 + 32 * d + 16 * h, 16)],
                             jnp.float32)
            v = jnp.clip(p * (KS - 1), 0.0, KS - 1 - 1e-6)
            i0 = v.astype(jnp.int32)
            fr = v - i0.astype(jnp.float32)
            i1 = jnp.minimum(i0 + 1, KS - 1)
            wd.append((1.0 - fr, fr))
            idd.append((i0, i1))
        eid = (w * EPT + b * CH_E + 16 * h) + lane
        m = jnp.where(eid < E, 1.0, 0.0).astype(jnp.float32)
        return wd, idd, m

    def basis_idx(b, mv, iv, dv):
        for h in range(2):
            src = mv[pl.ds(16 * h, 16)]
            dv[pl.ds(16 * h, 16)] = mv[pl.ds(32 + 16 * h, 16)]
            wd, idd, m = spline(b, mv, h)
            for s in range(4):
                ki = idd[0][s & 1] * KS + idd[1][(s >> 1) & 1]
                plsc.store_scatter(iv, [lane * 4 + (64 * h + s)],
                                   src * K + ki)

    def compute(b, B):
        rv, mv = rows_v[B], meta_v[B]
        for h in range(2):
            wd, idd, m = spline(b, mv, h)
            wregs = [wd[0][s & 1] * wd[1][(s >> 1) & 1] * m
                     for s in range(4)]
            dvec = mv[pl.ds(32 + 16 * h, 16)]
            for le in range(16):
                e = 16 * h + le
                w0 = wregs[0][le]
                w1 = wregs[1][le]
                w2 = wregs[2][le]
                w3 = wregs[3][le]
                for v in range(F // 16):
                    sl = pl.ds(16 * v, 16)
                    acc = (rv[4 * e + 0, sl] * w0
                           + rv[4 * e + 1, sl] * w1
                           + rv[4 * e + 2, sl] * w2
                           + rv[4 * e + 3, sl] * w3)
                    msg_v[e, sl] = acc
                d = dvec[le]
                dr = lax.shift_right_logical(d, 7)
                dbase = lax.bitwise_and(d, 0x70)
                dlane = lax.bitwise_and(d, 0xF)
                sl_d = pl.ds(dbase, 16)
                deg_v[dr, sl_d] = deg_v[dr, sl_d] + jnp.where(
                    lane == dlane, m[le], 0.0)

    def body(b, B):
        B2 = 1 - B

        @pl.when(b + 1 < CHUNKS)
        def _():
            pltpu.make_async_copy(meta.at[mrow + b + 1], meta_v[B2],
                                  sem_m[B2]).wait()
            basis_idx(b + 1, meta_v[B2], idx_v[B2], dst_v[B2])
            pltpu.async_copy(table.at[idx_v[B2]], rows_v[B2], sem_g[B2])

        pltpu.make_async_copy(table.at[idx_v[B]], rows_v[B],
                              sem_g[B]).wait()
        compute(b, B)

        @pl.when(b + 2 < CHUNKS)
        def _():
            pltpu.async_copy(meta.at[mrow + b + 2], meta_v[B], sem_m[B])

        pltpu.sync_copy(msg_v, agg_sh.at[dst_v[B]], add=True)

    pltpu.sync_copy(meta.at[mrow], meta_v[0])
    basis_idx(0, meta_v[0], idx_v[0], dst_v[0])
    pltpu.async_copy(table.at[idx_v[0]], rows_v[0], sem_g[0])
    pltpu.async_copy(meta.at[mrow + 1], meta_v[1], sem_m[1])

    @pl.loop(0, CHUNKS // 2)
    def _(g):
        body(2 * g, 0)
        body(2 * g + 1, 1)

    degs_out = degs.at[w]
    pltpu.sync_copy(deg_v, degs_out)
    plsc.subcore_barrier()
    pltpu.sync_copy(agg_sh.at[pl.ds(sid * NPS, NPS)],
                    out.at[cid, pl.ds(sid * NPS, NPS)])


def kernel(features, edge_index, pseudo, weight, bias):
    f32 = jnp.float32

    feat_pad = jnp.pad(features, ((0, NPAD - N), (0, 0)))
    w2 = jnp.transpose(weight, (1, 0, 2)).reshape(F, K * F)

    pad = EPAD - E
    src2 = jnp.pad(edge_index[0], (0, pad)).reshape(ROWS, 32)
    dst2 = jnp.pad(edge_index[1], (0, pad)).reshape(ROWS, 32)
    p0 = lax.bitcast_convert_type(
        jnp.pad(pseudo[:, 0], (0, pad)).reshape(ROWS, 32), jnp.int32)
    p1 = lax.bitcast_convert_type(
        jnp.pad(pseudo[:, 1], (0, pad)).reshape(ROWS, 32), jnp.int32)
    meta = jnp.concatenate([src2, dst2, p0, p1], axis=1)
    zeros = jnp.zeros((NPS, F), f32)

    mm = pl.pallas_call(
        _mm_body,
        grid=(NB,),
        in_specs=[pl.BlockSpec((NPAD // NB, F), lambda m: (m, 0)),
                  pl.BlockSpec((F, K * F), lambda m: (0, 0))],
        out_specs=pl.BlockSpec((NPAD // NB, K * F), lambda m: (m, 0)),
        out_shape=jax.ShapeDtypeStruct((NPAD, K * F), f32),
    )
    table = mm(feat_pad, w2).reshape(NPAD * K, F)

    mesh = plsc.VectorSubcoreMesh(core_axis_name="c", subcore_axis_name="s")
    cp = pltpu.CompilerParams()
    if "needs_layout_passes" in pltpu.CompilerParams.__dataclass_fields__:
        cp = dataclasses.replace(cp, needs_layout_passes=False)
    sc = pl.kernel(
        _sc_edge_kernel,
        mesh=mesh,
        out_type=[jax.ShapeDtypeStruct((2, NAGG, F), f32),
                  jax.ShapeDtypeStruct((NTILES, DROWS, 128), f32)],
        scratch_types=(
            [pltpu.VMEM((128,), jnp.int32)] * 2
            + [pltpu.VMEM((CH_E,), jnp.int32)] * 2
            + [pltpu.VMEM((128,), jnp.int32)] * 2
            + [pltpu.VMEM((128, F), f32)] * 2
            + [pltpu.VMEM((CH_E, F), f32)]
            + [pltpu.VMEM((DROWS, 128), f32),
               pltpu.VMEM_SHARED((NAGG, F), f32)]
            + [pltpu.SemaphoreType.DMA] * 4
        ),
        compiler_params=cp,
    )
    parts, degp = sc(table, meta, zeros)
    degf = degp.reshape(NTILES, NAGG, 1)

    norm = pl.pallas_call(
        _norm_body,
        grid=(10,),
        in_specs=[pl.BlockSpec((2, N // 10, F), lambda i: (0, i, 0)),
                  pl.BlockSpec((NTILES, N // 10, 1), lambda i: (0, i, 0)),
                  pl.BlockSpec((1, F), lambda i: (0, 0))],
        out_specs=pl.BlockSpec((N // 10, F), lambda i: (i, 0)),
        out_shape=jax.ShapeDtypeStruct((N, F), f32),
    )
    return norm(parts, degf, bias.reshape(1, F))


# speedup vs baseline: 3.3788x; 1.1274x over previous
import dataclasses

import jax
import jax.numpy as jnp
from jax import lax
from jax.experimental import pallas as pl
from jax.experimental.pallas import tpu as pltpu
from jax.experimental.pallas import tpu_sc as plsc

N = 10000
E = 320000
F = 128
K = 25
KS = 5
W2C = K * 64

NPAD = 10240
NB = 40
CH_E = 32
NTILES = 32
CHUNKS = 316
EPT = CH_E * CHUNKS
EPAD = EPT * NTILES
ROWS = EPAD // 32
NAGG = 10240
NPS = NAGG // 16
DROWS = NAGG // 128


def _mm_body(f_ref, wlo_ref, whi_ref, o_ref):
    f = f_ref[...]
    lo = jnp.dot(f, wlo_ref[...], preferred_element_type=jnp.float32)
    hi = jnp.dot(f, whi_ref[...], preferred_element_type=jnp.float32)
    lo16 = lax.bitcast_convert_type(lo.astype(jnp.bfloat16),
                                    jnp.uint16).astype(jnp.uint32)
    hi16 = lax.bitcast_convert_type(hi.astype(jnp.bfloat16),
                                    jnp.uint16).astype(jnp.uint32)
    word = jnp.bitwise_or(jnp.left_shift(hi16, 16), lo16)
    o_ref[...] = lax.bitcast_convert_type(word, jnp.float32)


def _norm_body(p_ref, d_ref, b_ref, o_ref):
    msg = p_ref[0] + p_ref[1]
    deg = jnp.sum(d_ref[...], axis=0)
    o_ref[...] = msg / jnp.maximum(deg, 1.0) + b_ref[...]


def _sc_edge_kernel(table, meta, zeros, out, degs,
                    meta_v0, meta_v1, dst_v0, dst_v1, idx_v0, idx_v1,
                    rows_v0, rows_v1, msg_v0, msg_v1, deg_va, deg_vb,
                    agg_sh, sg0, sg1, sm0, sm1, ss0, ss1):
    meta_v = (meta_v0, meta_v1)
    dst_v = (dst_v0, dst_v1)
    idx_v = (idx_v0, idx_v1)
    rows_v = (rows_v0, rows_v1)
    msg_v = (msg_v0, msg_v1)
    deg_v = (deg_va, deg_vb)
    sem_g = (sg0, sg1)
    sem_m = (sm0, sm1)
    sem_s = (ss0, ss1)

    cid = lax.axis_index("c")
    sid = lax.axis_index("s")
    w = sid * 2 + cid
    mrow = w * CHUNKS

    lane = lax.iota(jnp.int32, 16)

    pltpu.sync_copy(zeros, agg_sh.at[pl.ds(sid * NPS, NPS)])
    pltpu.sync_copy(zeros.at[pl.ds(0, DROWS)], deg_va)
    pltpu.sync_copy(zeros.at[pl.ds(0, DROWS)], deg_vb)
    plsc.subcore_barrier()

    def spline(b, mv, h):
        wd = []
        idd = []
        for d in range(2):
            p = plsc.bitcast(mv[pl.ds(64 + 32 * d + 16 * h, 16)],
                             jnp.float32)
            v = jnp.clip(p * (KS - 1), 0.0, KS - 1 - 1e-6)
            i0 = v.astype(jnp.int32)
            fr = v - i0.astype(jnp.float32)
            i1 = jnp.minimum(i0 + 1, KS - 1)
            wd.append((1.0 - fr, fr))
            idd.append((i0, i1))
        eid = (w * EPT + b * CH_E + 16 * h) + lane
        m = jnp.where(eid < E, 1.0, 0.0).astype(jnp.float32)
        return wd, idd, m

    def basis_idx(b, mv, iv, dv):
        for h in range(2):
            src = mv[pl.ds(16 * h, 16)]
            dv[pl.ds(16 * h, 16)] = mv[pl.ds(32 + 16 * h, 16)]
            wd, idd, m = spline(b, mv, h)
            for s in range(4):
                ki = idd[0][s & 1] * KS + idd[1][(s >> 1) & 1]
                plsc.store_scatter(iv, [lane * 4 + (64 * h + s)],
                                   src * K + ki)

    def compute(b, B):
        rv, mv = rows_v[B], meta_v[B]
        msg = msg_v[B]
        for h in range(2):
            wd, idd, m = spline(b, mv, h)
            wregs = [wd[0][s & 1] * wd[1][(s >> 1) & 1] * m
                     for s in range(4)]
            dvec = mv[pl.ds(32 + 16 * h, 16)]
            for le in range(16):
                e = 16 * h + le
                ws = [wregs[s][le] for s in range(4)]
                for v in range(4):
                    sl = pl.ds(16 * v, 16)
                    acc_lo = None
                    acc_hi = None
                    for s in range(4):
                        pk = plsc.bitcast(rv[4 * e + s, sl], jnp.bfloat16)
                        lo, hi = plsc.unpack(
                            pk, format=plsc.PackFormat.INTERLEAVED)
                        tl = lo * ws[s]
                        th = hi * ws[s]
                        acc_lo = tl if acc_lo is None else acc_lo + tl
                        acc_hi = th if acc_hi is None else acc_hi + th
                    msg[e, sl] = acc_lo
                    msg[e, pl.ds(64 + 16 * v, 16)] = acc_hi
                dg = deg_v[le % 2]
                d = dvec[le]
                dr = lax.shift_right_logical(d, 7)
                dbase = lax.bitwise_and(d, 0x70)
                dlane = lax.bitwise_and(d, 0xF)
                sl_d = pl.ds(dbase, 16)
                dg[dr, sl_d] = dg[dr, sl_d] + jnp.where(
                    lane == dlane, m[le], 0.0)

    def body(b, B):
        B2 = 1 - B

        @pl.when(b >= 1)
        def _():
            pltpu.make_async_copy(msg_v[B2], agg_sh.at[dst_v[B2]],
                                  sem_s[B2]).wait()

        @pl.when(b + 1 < CHUNKS)
        def _():
            pltpu.make_async_copy(meta.at[mrow + b + 1], meta_v[B2],
                                  sem_m[B2]).wait()
            basis_idx(b + 1, meta_v[B2], idx_v[B2], dst_v[B2])
            pltpu.async_copy(table.at[idx_v[B2]], rows_v[B2], sem_g[B2])

        pltpu.make_async_copy(table.at[idx_v[B]], rows_v[B],
                              sem_g[B]).wait()
        compute(b, B)

        @pl.when(b + 2 < CHUNKS)
        def _():
            pltpu.async_copy(meta.at[mrow + b + 2], meta_v[B], sem_m[B])

        pltpu.async_copy(msg_v[B], agg_sh.at[dst_v[B]], sem_s[B], add=True)

    pltpu.sync_copy(meta.at[mrow], meta_v[0])
    basis_idx(0, meta_v[0], idx_v[0], dst_v[0])
    pltpu.async_copy(table.at[idx_v[0]], rows_v[0], sem_g[0])
    pltpu.async_copy(meta.at[mrow + 1], meta_v[1], sem_m[1])

    @pl.loop(0, CHUNKS // 2)
    def _(g):
        body(2 * g, 0)
        body(2 * g + 1, 1)

    pltpu.make_async_copy(msg_v[1], agg_sh.at[dst_v[1]], sem_s[1]).wait()

    @pl.loop(0, DROWS)
    def _(r):
        for g in range(8):
            sl = pl.ds(16 * g, 16)
            deg_va[r, sl] = deg_va[r, sl] + deg_vb[r, sl]

    pltpu.sync_copy(deg_va, degs.at[w])
    plsc.subcore_barrier()
    pltpu.sync_copy(agg_sh.at[pl.ds(sid * NPS, NPS)],
                    out.at[cid, pl.ds(sid * NPS, NPS)])


def kernel(features, edge_index, pseudo, weight, bias):
    f32 = jnp.float32

    feat_pad = jnp.pad(features, ((0, NPAD - N), (0, 0)))
    w3 = jnp.transpose(weight, (1, 0, 2))
    wlo = w3[:, :, :64].reshape(F, W2C)
    whi = w3[:, :, 64:].reshape(F, W2C)

    pad = EPAD - E
    src2 = jnp.pad(edge_index[0], (0, pad)).reshape(ROWS, 32)
    dst2 = jnp.pad(edge_index[1], (0, pad)).reshape(ROWS, 32)
    p0 = lax.bitcast_convert_type(
        jnp.pad(pseudo[:, 0], (0, pad)).reshape(ROWS, 32), jnp.int32)
    p1 = lax.bitcast_convert_type(
        jnp.pad(pseudo[:, 1], (0, pad)).reshape(ROWS, 32), jnp.int32)
    meta = jnp.concatenate([src2, dst2, p0, p1], axis=1)
    zeros = jnp.zeros((NPS, F), f32)

    mm = pl.pallas_call(
        _mm_body,
        grid=(NB,),
        in_specs=[pl.BlockSpec((NPAD // NB, F), lambda m: (m, 0)),
                  pl.BlockSpec((F, W2C), lambda m: (0, 0)),
                  pl.BlockSpec((F, W2C), lambda m: (0, 0))],
        out_specs=pl.BlockSpec((NPAD // NB, W2C), lambda m: (m, 0)),
        out_shape=jax.ShapeDtypeStruct((NPAD, W2C), f32),
    )
    table = mm(feat_pad, wlo, whi).reshape(NPAD * K, 64)

    mesh = plsc.VectorSubcoreMesh(core_axis_name="c", subcore_axis_name="s")
    cp = pltpu.CompilerParams()
    fields = pltpu.CompilerParams.__dataclass_fields__
    if "needs_layout_passes" in fields:
        cp = dataclasses.replace(cp, needs_layout_passes=False)
    if "use_tc_tiling_on_sc" in fields:
        cp = dataclasses.replace(cp, use_tc_tiling_on_sc=False)
    sc = pl.kernel(
        _sc_edge_kernel,
        mesh=mesh,
        out_type=[jax.ShapeDtypeStruct((2, NAGG, F), f32),
                  jax.ShapeDtypeStruct((NTILES, DROWS, 128), f32)],
        scratch_types=(
            [pltpu.VMEM((128,), jnp.int32)] * 2
            + [pltpu.VMEM((CH_E,), jnp.int32)] * 2
            + [pltpu.VMEM((128,), jnp.int32)] * 2
            + [pltpu.VMEM((128, 64), f32)] * 2
            + [pltpu.VMEM((CH_E, F), f32)] * 2
            + [pltpu.VMEM((DROWS, 128), f32)] * 2
            + [pltpu.VMEM_SHARED((NAGG, F), f32)]
            + [pltpu.SemaphoreType.DMA] * 6
        ),
        compiler_params=cp,
    )
    parts, degp = sc(table, meta, zeros)
    degf = degp.reshape(NTILES, NAGG, 1)

    norm = pl.pallas_call(
        _norm_body,
        grid=(10,),
        in_specs=[pl.BlockSpec((2, N // 10, F), lambda i: (0, i, 0)),
                  pl.BlockSpec((NTILES, N // 10, 1), lambda i: (0, i, 0)),
                  pl.BlockSpec((1, F), lambda i: (0, 0))],
        out_specs=pl.BlockSpec((N // 10, F), lambda i: (i, 0)),
        out_shape=jax.ShapeDtypeStruct((N, F), f32),
    )
    return norm(parts, degf, bias.reshape(1, F))


# speedup vs baseline: 3.9932x; 1.1818x over previous
import dataclasses

import jax
import jax.numpy as jnp
from jax import lax
from jax.experimental import pallas as pl
from jax.experimental.pallas import tpu as pltpu
from jax.experimental.pallas import tpu_sc as plsc

N = 10000
E = 320000
F = 128
K = 25
KS = 5
W2C = K * 64

NPAD = 10240
NB = 40
CH_E = 32
NTILES = 32
CHUNKS = 316
EPT = CH_E * CHUNKS
EPAD = EPT * NTILES
ROWS = EPAD // 32
NAGG = 10240
NPS = NAGG // 16
DROWS = NAGG // 128


def _mm_body(f_ref, wlo_ref, whi_ref, o_ref):
    f = f_ref[...]
    lo = jnp.dot(f, wlo_ref[...], preferred_element_type=jnp.float32)
    hi = jnp.dot(f, whi_ref[...], preferred_element_type=jnp.float32)
    lo16 = lax.bitcast_convert_type(lo.astype(jnp.bfloat16),
                                    jnp.uint16).astype(jnp.uint32)
    hi16 = lax.bitcast_convert_type(hi.astype(jnp.bfloat16),
                                    jnp.uint16).astype(jnp.uint32)
    word = jnp.bitwise_or(jnp.left_shift(hi16, 16), lo16)
    o_ref[...] = lax.bitcast_convert_type(word, jnp.float32)


def _norm_body(p_ref, d_ref, b_ref, o_ref):
    msg = p_ref[0] + p_ref[1]
    deg = jnp.sum(d_ref[...], axis=0)
    o_ref[...] = msg / jnp.maximum(deg, 1.0) + b_ref[...]


def _sc_edge_kernel(table, meta, zeros, out, degs,
                    meta_v0, meta_v1, dst_v0, dst_v1, idx_v0, idx_v1,
                    rows_v0, rows_v1, msg_v0, msg_v1, deg_va, deg_vb,
                    agg_sh, sg0, sg1, sm0, sm1, ss0, ss1):
    meta_v = (meta_v0, meta_v1)
    dst_v = (dst_v0, dst_v1)
    idx_v = (idx_v0, idx_v1)
    rows_v = (rows_v0, rows_v1)
    msg_v = (msg_v0, msg_v1)
    deg_v = (deg_va, deg_vb)
    sem_g = (sg0, sg1)
    sem_m = (sm0, sm1)
    sem_s = (ss0, ss1)

    cid = lax.axis_index("c")
    sid = lax.axis_index("s")
    w = sid * 2 + cid
    mrow = w * CHUNKS

    lane = lax.iota(jnp.int32, 16)
    fone = lane.astype(jnp.float32) * 0.0 + 1.0

    pltpu.sync_copy(zeros, agg_sh.at[pl.ds(sid * NPS, NPS)])
    pltpu.sync_copy(zeros.at[pl.ds(0, DROWS)], deg_va)
    pltpu.sync_copy(zeros.at[pl.ds(0, DROWS)], deg_vb)
    plsc.subcore_barrier()

    def spline(b, mv, h):
        wd = []
        idd = []
        for d in range(2):
            p = plsc.bitcast(mv[pl.ds(64 + 32 * d + 16 * h, 16)],
                             jnp.float32)
            v = jnp.clip(p * (KS - 1), 0.0, KS - 1 - 1e-6)
            i0 = v.astype(jnp.int32)
            fr = v - i0.astype(jnp.float32)
            i1 = jnp.minimum(i0 + 1, KS - 1)
            wd.append((1.0 - fr, fr))
            idd.append((i0, i1))
        eid = (w * EPT + b * CH_E + 16 * h) + lane
        m = jnp.where(eid < E, 1.0, 0.0).astype(jnp.float32)
        return wd, idd, m

    def basis_idx(b, mv, iv, dv):
        for h in range(2):
            src = mv[pl.ds(16 * h, 16)]
            dv[pl.ds(16 * h, 16)] = mv[pl.ds(32 + 16 * h, 16)]
            wd, idd, m = spline(b, mv, h)
            for s in range(4):
                ki = idd[0][s & 1] * KS + idd[1][(s >> 1) & 1]
                plsc.store_scatter(iv, [lane * 4 + (64 * h + s)],
                                   src * K + ki)

    def compute(b, B):
        rv, mv = rows_v[B], meta_v[B]
        msg = msg_v[B]
        for h in range(2):
            wd, idd, m = spline(b, mv, h)
            wregs = [wd[0][s & 1] * wd[1][(s >> 1) & 1] * m
                     for s in range(4)]
            dvec = mv[pl.ds(32 + 16 * h, 16)]
            for le in range(16):
                e = 16 * h + le
                ws = []
                for s in range(4):
                    wvec = fone * wregs[s][le]
                    ws.append(plsc.pack(
                        wvec, wvec, format=plsc.PackFormat.INTERLEAVED))
                for v in range(4):
                    sl = pl.ds(16 * v, 16)
                    acc = None
                    for s in range(4):
                        pk = plsc.bitcast(rv[4 * e + s, sl], jnp.bfloat16)
                        t = pk * ws[s]
                        acc = t if acc is None else acc + t
                    lo, hi = plsc.unpack(
                        acc, format=plsc.PackFormat.INTERLEAVED)
                    msg[e, sl] = lo
                    msg[e, pl.ds(64 + 16 * v, 16)] = hi
                dg = deg_v[le % 2]
                d = dvec[le]
                dr = lax.shift_right_logical(d, 7)
                dbase = lax.bitwise_and(d, 0x70)
                dlane = lax.bitwise_and(d, 0xF)
                sl_d = pl.ds(dbase, 16)
                dg[dr, sl_d] = dg[dr, sl_d] + jnp.where(
                    lane == dlane, m[le], 0.0)

    def body(b, B):
        B2 = 1 - B

        @pl.when(b >= 1)
        def _():
            pltpu.make_async_copy(msg_v[B2], agg_sh.at[dst_v[B2]],
                                  sem_s[B2]).wait()

        @pl.when(b + 1 < CHUNKS)
        def _():
            pltpu.make_async_copy(meta.at[mrow + b + 1], meta_v[B2],
                                  sem_m[B2]).wait()
            basis_idx(b + 1, meta_v[B2], idx_v[B2], dst_v[B2])
            pltpu.async_copy(table.at[idx_v[B2]], rows_v[B2], sem_g[B2])

        pltpu.make_async_copy(table.at[idx_v[B]], rows_v[B],
                              sem_g[B]).wait()
        compute(b, B)

        @pl.when(b + 2 < CHUNKS)
        def _():
            pltpu.async_copy(meta.at[mrow + b + 2], meta_v[B], sem_m[B])

        pltpu.async_copy(msg_v[B], agg_sh.at[dst_v[B]], sem_s[B], add=True)

    pltpu.sync_copy(meta.at[mrow], meta_v[0])
    basis_idx(0, meta_v[0], idx_v[0], dst_v[0])
    pltpu.async_copy(table.at[idx_v[0]], rows_v[0], sem_g[0])
    pltpu.async_copy(meta.at[mrow + 1], meta_v[1], sem_m[1])

    @pl.loop(0, CHUNKS // 2)
    def _(g):
        body(2 * g, 0)
        body(2 * g + 1, 1)

    pltpu.make_async_copy(msg_v[1], agg_sh.at[dst_v[1]], sem_s[1]).wait()

    @pl.loop(0, DROWS)
    def _(r):
        for g in range(8):
            sl = pl.ds(16 * g, 16)
            deg_va[r, sl] = deg_va[r, sl] + deg_vb[r, sl]

    pltpu.sync_copy(deg_va, degs.at[w])
    plsc.subcore_barrier()
    pltpu.sync_copy(agg_sh.at[pl.ds(sid * NPS, NPS)],
                    out.at[cid, pl.ds(sid * NPS, NPS)])


def kernel(features, edge_index, pseudo, weight, bias):
    f32 = jnp.float32

    feat_pad = jnp.pad(features, ((0, NPAD - N), (0, 0)))
    w3 = jnp.transpose(weight, (1, 0, 2))
    wlo = w3[:, :, :64].reshape(F, W2C)
    whi = w3[:, :, 64:].reshape(F, W2C)

    pad = EPAD - E
    src2 = jnp.pad(edge_index[0], (0, pad)).reshape(ROWS, 32)
    dst2 = jnp.pad(edge_index[1], (0, pad)).reshape(ROWS, 32)
    p0 = lax.bitcast_convert_type(
        jnp.pad(pseudo[:, 0], (0, pad)).reshape(ROWS, 32), jnp.int32)
    p1 = lax.bitcast_convert_type(
        jnp.pad(pseudo[:, 1], (0, pad)).reshape(ROWS, 32), jnp.int32)
    meta = jnp.concatenate([src2, dst2, p0, p1], axis=1)
    zeros = jnp.zeros((NPS, F), f32)

    mm = pl.pallas_call(
        _mm_body,
        grid=(NB,),
        in_specs=[pl.BlockSpec((NPAD // NB, F), lambda m: (m, 0)),
                  pl.BlockSpec((F, W2C), lambda m: (0, 0)),
                  pl.BlockSpec((F, W2C), lambda m: (0, 0))],
        out_specs=pl.BlockSpec((NPAD // NB, W2C), lambda m: (m, 0)),
        out_shape=jax.ShapeDtypeStruct((NPAD, W2C), f32),
    )
    table = mm(feat_pad, wlo, whi).reshape(NPAD * K, 64)

    mesh = plsc.VectorSubcoreMesh(core_axis_name="c", subcore_axis_name="s")
    cp = pltpu.CompilerParams()
    fields = pltpu.CompilerParams.__dataclass_fields__
    if "needs_layout_passes" in fields:
        cp = dataclasses.replace(cp, needs_layout_passes=False)
    if "use_tc_tiling_on_sc" in fields:
        cp = dataclasses.replace(cp, use_tc_tiling_on_sc=False)
    sc = pl.kernel(
        _sc_edge_kernel,
        mesh=mesh,
        out_type=[jax.ShapeDtypeStruct((2, NAGG, F), f32),
                  jax.ShapeDtypeStruct((NTILES, DROWS, 128), f32)],
        scratch_types=(
            [pltpu.VMEM((128,), jnp.int32)] * 2
            + [pltpu.VMEM((CH_E,), jnp.int32)] * 2
            + [pltpu.VMEM((128,), jnp.int32)] * 2
            + [pltpu.VMEM((128, 64), f32)] * 2
            + [pltpu.VMEM((CH_E, F), f32)] * 2
            + [pltpu.VMEM((DROWS, 128), f32)] * 2
            + [pltpu.VMEM_SHARED((NAGG, F), f32)]
            + [pltpu.SemaphoreType.DMA] * 6
        ),
        compiler_params=cp,
    )
    parts, degp = sc(table, meta, zeros)
    degf = degp.reshape(NTILES, NAGG, 1)

    norm = pl.pallas_call(
        _norm_body,
        grid=(10,),
        in_specs=[pl.BlockSpec((2, N // 10, F), lambda i: (0, i, 0)),
                  pl.BlockSpec((NTILES, N // 10, 1), lambda i: (0, i, 0)),
                  pl.BlockSpec((1, F), lambda i: (0, 0))],
        out_specs=pl.BlockSpec((N // 10, F), lambda i: (i, 0)),
        out_shape=jax.ShapeDtypeStruct((N, F), f32),
    )
    return norm(parts, degf, bias.reshape(1, F))


# speedup vs baseline: 4.8084x; 1.2041x over previous
import dataclasses

import jax
import jax.numpy as jnp
from jax import lax
from jax.experimental import pallas as pl
from jax.experimental.pallas import tpu as pltpu
from jax.experimental.pallas import tpu_sc as plsc

N = 10000
E = 320000
F = 128
K = 25
KS = 5
W2C = K * 64

NPAD = 10240
NB = 40
CH_E = 32
NTILES = 32
CHUNKS = 316
EPT = CH_E * CHUNKS
EPAD = EPT * NTILES
ROWS = EPAD // 32
NAGG = 10240
NPS = NAGG // 16
DROWS = NAGG // 128


def _mm_body(f_ref, wlo_ref, whi_ref, o_ref):
    f = f_ref[...]
    lo = jnp.dot(f, wlo_ref[...], preferred_element_type=jnp.float32)
    hi = jnp.dot(f, whi_ref[...], preferred_element_type=jnp.float32)
    lo16 = lax.bitcast_convert_type(lo.astype(jnp.bfloat16),
                                    jnp.uint16).astype(jnp.uint32)
    hi16 = lax.bitcast_convert_type(hi.astype(jnp.bfloat16),
                                    jnp.uint16).astype(jnp.uint32)
    word = jnp.bitwise_or(jnp.left_shift(hi16, 16), lo16)
    o_ref[...] = lax.bitcast_convert_type(word, jnp.float32)


def _degsum_body(d_ref, o_ref):
    o_ref[...] = jnp.sum(d_ref[...], axis=0)


def _norm_body(p_ref, d_ref, b_ref, o_ref):
    msg = p_ref[0] + p_ref[1]
    deg = d_ref[...]
    o_ref[...] = msg / jnp.maximum(deg, 1.0) + b_ref[...]


def _sc_edge_kernel(table, meta, zeros, zerod, out, degs,
                    meta_v0, meta_v1, dst_v0, dst_v1, idx_v0, idx_v1,
                    rows_v0, rows_v1, msg_v0, msg_v1, deg_va, deg_vb,
                    agg_sh, sg0, sg1, sm0, sm1, ss0, ss1):
    meta_v = (meta_v0, meta_v1)
    dst_v = (dst_v0, dst_v1)
    idx_v = (idx_v0, idx_v1)
    rows_v = (rows_v0, rows_v1)
    msg_v = (msg_v0, msg_v1)
    deg_v = (deg_va, deg_vb)
    sem_g = (sg0, sg1)
    sem_m = (sm0, sm1)
    sem_s = (ss0, ss1)

    cid = lax.axis_index("c")
    sid = lax.axis_index("s")
    w = sid * 2 + cid
    mrow = w * CHUNKS

    lane = lax.iota(jnp.int32, 16)
    fone = lane.astype(jnp.float32) * 0.0 + 1.0

    pltpu.sync_copy(zeros, agg_sh.at[pl.ds(sid * NPS, NPS)])
    pltpu.sync_copy(zerod, deg_va)
    pltpu.sync_copy(zerod, deg_vb)
    plsc.subcore_barrier()

    def spline(b, mv, h):
        wd = []
        idd = []
        for d in range(2):
            p = plsc.bitcast(mv[pl.ds(64 + 32 * d + 16 * h, 16)],
                             jnp.float32)
            v = jnp.clip(p * (KS - 1), 0.0, KS - 1 - 1e-6)
            i0 = v.astype(jnp.int32)
            fr = v - i0.astype(jnp.float32)
            i1 = jnp.minimum(i0 + 1, KS - 1)
            wd.append((1.0 - fr, fr))
            idd.append((i0, i1))
        eid = (w * EPT + b * CH_E + 16 * h) + lane
        m = jnp.where(eid < E, 1.0, 0.0).astype(jnp.float32)
        return wd, idd, m

    def basis_idx(b, mv, iv, dv):
        for h in range(2):
            src = mv[pl.ds(16 * h, 16)]
            dv[pl.ds(16 * h, 16)] = mv[pl.ds(32 + 16 * h, 16)]
            wd, idd, m = spline(b, mv, h)
            for s in range(4):
                ki = idd[0][s & 1] * KS + idd[1][(s >> 1) & 1]
                plsc.store_scatter(iv, [lane * 4 + (64 * h + s)],
                                   src * K + ki)

    def compute(b, B):
        rv, mv = rows_v[B], meta_v[B]
        msg = msg_v[B]
        for h in range(2):
            wd, idd, m = spline(b, mv, h)
            wregs = [wd[0][s & 1] * wd[1][(s >> 1) & 1] * m
                     for s in range(4)]
            dvec = mv[pl.ds(32 + 16 * h, 16)]
            for le in range(16):
                e = 16 * h + le
                ws = []
                for s in range(4):
                    wvec = fone * wregs[s][le]
                    ws.append(plsc.pack(
                        wvec, wvec, format=plsc.PackFormat.INTERLEAVED))
                for v in range(4):
                    sl = pl.ds(16 * v, 16)
                    acc = None
                    for s in range(4):
                        pk = plsc.bitcast(rv[4 * e + s, sl], jnp.bfloat16)
                        t = pk * ws[s]
                        acc = t if acc is None else acc + t
                    lo, hi = plsc.unpack(
                        acc, format=plsc.PackFormat.INTERLEAVED)
                    msg[e, sl] = lo
                    msg[e, pl.ds(64 + 16 * v, 16)] = hi
                dg = deg_v[le % 2]
                d = dvec[le]
                dbase = lax.bitwise_and(d, 0x3FF0)
                dlane = lax.bitwise_and(d, 0xF)
                sl_d = pl.ds(dbase, 16)
                dg[sl_d] = dg[sl_d] + jnp.where(lane == dlane, m[le], 0.0)

    def body(b, B):
        B2 = 1 - B

        @pl.when(b >= 1)
        def _():
            pltpu.make_async_copy(msg_v[B2], agg_sh.at[dst_v[B2]],
                                  sem_s[B2]).wait()

        @pl.when(b + 1 < CHUNKS)
        def _():
            pltpu.make_async_copy(meta.at[mrow + b + 1], meta_v[B2],
                                  sem_m[B2]).wait()
            basis_idx(b + 1, meta_v[B2], idx_v[B2], dst_v[B2])
            pltpu.async_copy(table.at[idx_v[B2]], rows_v[B2], sem_g[B2])

        pltpu.make_async_copy(table.at[idx_v[B]], rows_v[B],
                              sem_g[B]).wait()
        compute(b, B)

        @pl.when(b + 2 < CHUNKS)
        def _():
            pltpu.async_copy(meta.at[mrow + b + 2], meta_v[B], sem_m[B])

        pltpu.async_copy(msg_v[B], agg_sh.at[dst_v[B]], sem_s[B], add=True)

    pltpu.sync_copy(meta.at[mrow], meta_v[0])
    basis_idx(0, meta_v[0], idx_v[0], dst_v[0])
    pltpu.async_copy(table.at[idx_v[0]], rows_v[0], sem_g[0])
    pltpu.async_copy(meta.at[mrow + 1], meta_v[1], sem_m[1])

    @pl.loop(0, CHUNKS // 2)
    def _(g):
        body(2 * g, 0)
        body(2 * g + 1, 1)

    pltpu.make_async_copy(msg_v[1], agg_sh.at[dst_v[1]], sem_s[1]).wait()

    pltpu.sync_copy(deg_va, degs.at[0, w])
    pltpu.sync_copy(deg_vb, degs.at[1, w])
    plsc.subcore_barrier()
    pltpu.sync_copy(agg_sh.at[pl.ds(sid * NPS, NPS)],
                    out.at[cid, pl.ds(sid * NPS, NPS)])


def kernel(features, edge_index, pseudo, weight, bias):
    f32 = jnp.float32

    feat_pad = jnp.pad(features, ((0, NPAD - N), (0, 0)))
    w3 = jnp.transpose(weight, (1, 0, 2))
    wlo = w3[:, :, :64].reshape(F, W2C)
    whi = w3[:, :, 64:].reshape(F, W2C)

    pad = EPAD - E
    src2 = jnp.pad(edge_index[0], (0, pad)).reshape(ROWS, 32)
    dst2 = jnp.pad(edge_index[1], (0, pad)).reshape(ROWS, 32)
    p0 = lax.bitcast_convert_type(
        jnp.pad(pseudo[:, 0], (0, pad)).reshape(ROWS, 32), jnp.int32)
    p1 = lax.bitcast_convert_type(
        jnp.pad(pseudo[:, 1], (0, pad)).reshape(ROWS, 32), jnp.int32)
    meta = jnp.concatenate([src2, dst2, p0, p1], axis=1)
    zeros = jnp.zeros((NPS, F), f32)
    zerod = jnp.zeros((NAGG,), f32)

    mm = pl.pallas_call(
        _mm_body,
        grid=(NB,),
        in_specs=[pl.BlockSpec((NPAD // NB, F), lambda m: (m, 0)),
                  pl.BlockSpec((F, W2C), lambda m: (0, 0)),
                  pl.BlockSpec((F, W2C), lambda m: (0, 0))],
        out_specs=pl.BlockSpec((NPAD // NB, W2C), lambda m: (m, 0)),
        out_shape=jax.ShapeDtypeStruct((NPAD, W2C), f32),
    )
    table = mm(feat_pad.astype(jnp.bfloat16), wlo.astype(jnp.bfloat16),
               whi.astype(jnp.bfloat16)).reshape(NPAD * K, 64)

    mesh = plsc.VectorSubcoreMesh(core_axis_name="c", subcore_axis_name="s")
    cp = pltpu.CompilerParams()
    fields = pltpu.CompilerParams.__dataclass_fields__
    if "needs_layout_passes" in fields:
        cp = dataclasses.replace(cp, needs_layout_passes=False)
    if "use_tc_tiling_on_sc" in fields:
        cp = dataclasses.replace(cp, use_tc_tiling_on_sc=False)
    sc = pl.kernel(
        _sc_edge_kernel,
        mesh=mesh,
        out_type=[jax.ShapeDtypeStruct((2, NAGG, F), f32),
                  jax.ShapeDtypeStruct((2, NTILES, NAGG), f32)],
        scratch_types=(
            [pltpu.VMEM((128,), jnp.int32)] * 2
            + [pltpu.VMEM((CH_E,), jnp.int32)] * 2
            + [pltpu.VMEM((128,), jnp.int32)] * 2
            + [pltpu.VMEM((128, 64), f32)] * 2
            + [pltpu.VMEM((CH_E, F), f32)] * 2
            + [pltpu.VMEM((NAGG,), f32)] * 2
            + [pltpu.VMEM_SHARED((NAGG, F), f32)]
            + [pltpu.SemaphoreType.DMA] * 6
        ),
        compiler_params=cp,
    )
    parts, degp = sc(table, meta, zeros, zerod)

    degsum = pl.pallas_call(
        _degsum_body,
        grid=(1,),
        in_specs=[pl.BlockSpec((2 * NTILES, NAGG), lambda i: (0, 0))],
        out_specs=pl.BlockSpec((NAGG,), lambda i: (0,)),
        out_shape=jax.ShapeDtypeStruct((NAGG,), f32),
    )
    degf = degsum(degp.reshape(2 * NTILES, NAGG))[:, None]

    norm = pl.pallas_call(
        _norm_body,
        grid=(10,),
        in_specs=[pl.BlockSpec((2, N // 10, F), lambda i: (0, i, 0)),
                  pl.BlockSpec((N // 10, 1), lambda i: (i, 0)),
                  pl.BlockSpec((1, F), lambda i: (0, 0))],
        out_specs=pl.BlockSpec((N // 10, F), lambda i: (i, 0)),
        out_shape=jax.ShapeDtypeStruct((N, F), f32),
    )
    return norm(parts, degf, bias.reshape(1, F))
